# Initial kernel scaffold; baseline (speedup 1.0000x reference)
import jax
import jax.numpy as jnp
from jax.experimental import pallas as pl


def kernel(in_feat, edge_index, W1_self, W1_neigh, b1, W2_self, W2_neigh, b2):
    raise NotImplementedError("write your pallas kernel here")


# speedup vs baseline: 7.8851x; 7.8851x over previous
import functools

import jax
import jax.numpy as jnp
from jax import lax
from jax.experimental import pallas as pl
from jax.experimental.pallas import tpu as pltpu
from jax.experimental.pallas import tpu_sc as plsc

N = 10000
E = 320000
D_IN = 128
D_HID = 150
D_OUT = 128
DF1 = 136

NC, NS = 2, 16
NW = NC * NS
EW = E // NW
K = 80
NCH = EW // K
RPS = N // NS

_MESH = plsc.VectorSubcoreMesh(
    core_axis_name="c", subcore_axis_name="s", num_cores=NC, num_subcores=NS)


def _make_agg(DF):
  @functools.partial(
      pl.kernel,
      mesh=_MESH,
      compiler_params=pltpu.CompilerParams(use_tc_tiling_on_sc=False),
      out_type=jax.ShapeDtypeStruct((NC, N, DF), jnp.float32),
      scratch_types=[
          pltpu.VMEM((NCH, K), jnp.int32),
          pltpu.VMEM((NCH, K), jnp.int32),
          pltpu.VMEM((K, DF), jnp.float32),
          pltpu.VMEM_SHARED((N, DF), jnp.float32),
          pltpu.SemaphoreType.DMA,
      ],
  )
  def agg(feat_hbm, src_hbm, dst_hbm, zeros_hbm, out_hbm,
          src_v, dst_v, rows_v, acc, sem):
    c = lax.axis_index("c")
    s = lax.axis_index("s")
    wid = s * NC + c
    base = s * RPS
    pltpu.sync_copy(zeros_hbm.at[pl.ds(base, RPS)], acc.at[pl.ds(base, RPS)])
    pltpu.sync_copy(src_hbm.at[wid], src_v)
    pltpu.sync_copy(dst_hbm.at[wid], dst_v)
    plsc.subcore_barrier()

    def body(j, carry):
      pltpu.async_copy(feat_hbm.at[src_v.at[j]], rows_v, sem).wait()
      pltpu.sync_copy(rows_v, acc.at[dst_v.at[j]], add=True)
      return carry

    lax.fori_loop(0, NCH, body, 0)
    plsc.subcore_barrier()
    pltpu.sync_copy(acc.at[pl.ds(base, RPS)], out_hbm.at[c, pl.ds(base, RPS)])

  return agg


_AGG1 = _make_agg(DF1)
_AGG2 = _make_agg(D_OUT)

_R = 1000


def _tc1(x, parts1, w1s, w1n, b1, w2s, w2n, b2):
  def body(x_ref, p_ref, w1s_ref, w1n_ref, b1_ref, w2s_ref, w2n_ref, b2_ref,
           p2_ref, s2_ref, inv_ref):
    aggext = p_ref[0] + p_ref[1]
    inv = 1.0 / jnp.maximum(aggext[:, D_IN:D_IN + 1], 1.0)
    hn = aggext[:, :D_IN] * inv
    h1 = jnp.maximum(
        jnp.dot(x_ref[...], w1s_ref[...], preferred_element_type=jnp.float32)
        + jnp.dot(hn, w1n_ref[...], preferred_element_type=jnp.float32)
        + b1_ref[...], 0.0)
    p2_ref[...] = jnp.dot(h1, w2n_ref[...], preferred_element_type=jnp.float32)
    s2_ref[...] = (jnp.dot(h1, w2s_ref[...], preferred_element_type=jnp.float32)
                   + b2_ref[...])
    inv_ref[...] = jnp.broadcast_to(inv, (_R, D_OUT))

  return pl.pallas_call(
      body,
      grid=(N // _R,),
      in_specs=[
          pl.BlockSpec((_R, D_IN), lambda i: (i, 0)),
          pl.BlockSpec((NC, _R, DF1), lambda i: (0, i, 0)),
          pl.BlockSpec((D_IN, D_HID), lambda i: (0, 0)),
          pl.BlockSpec((D_IN, D_HID), lambda i: (0, 0)),
          pl.BlockSpec((1, D_HID), lambda i: (0, 0)),
          pl.BlockSpec((D_HID, D_OUT), lambda i: (0, 0)),
          pl.BlockSpec((D_HID, D_OUT), lambda i: (0, 0)),
          pl.BlockSpec((1, D_OUT), lambda i: (0, 0)),
      ],
      out_specs=[
          pl.BlockSpec((_R, D_OUT), lambda i: (i, 0)),
          pl.BlockSpec((_R, D_OUT), lambda i: (i, 0)),
          pl.BlockSpec((_R, D_OUT), lambda i: (i, 0)),
      ],
      out_shape=[
          jax.ShapeDtypeStruct((N, D_OUT), jnp.float32),
          jax.ShapeDtypeStruct((N, D_OUT), jnp.float32),
          jax.ShapeDtypeStruct((N, D_OUT), jnp.float32),
      ],
  )(x, parts1, w1s, w1n, b1, w2s, w2n, b2)


def _tc2(s2, parts2, invb):
  def body(s2_ref, p_ref, inv_ref, o_ref):
    o_ref[...] = s2_ref[...] + (p_ref[0] + p_ref[1]) * inv_ref[...]

  return pl.pallas_call(
      body,
      grid=(N // _R,),
      in_specs=[
          pl.BlockSpec((_R, D_OUT), lambda i: (i, 0)),
          pl.BlockSpec((NC, _R, D_OUT), lambda i: (0, i, 0)),
          pl.BlockSpec((_R, D_OUT), lambda i: (i, 0)),
      ],
      out_specs=pl.BlockSpec((_R, D_OUT), lambda i: (i, 0)),
      out_shape=jax.ShapeDtypeStruct((N, D_OUT), jnp.float32),
  )(s2, parts2, invb)


def kernel(in_feat, edge_index, W1_self, W1_neigh, b1, W2_self, W2_neigh, b2):
  src = edge_index[0].astype(jnp.int32).reshape(NW, NCH, K)
  dst = edge_index[1].astype(jnp.int32).reshape(NW, NCH, K)
  feat_ext = jnp.concatenate(
      [in_feat,
       jnp.ones((N, 1), jnp.float32),
       jnp.zeros((N, DF1 - D_IN - 1), jnp.float32)], axis=1)
  zeros1 = jnp.zeros((N, DF1), jnp.float32)
  parts1 = _AGG1(feat_ext, src, dst, zeros1)
  p2, s2, invb = _tc1(in_feat, parts1, W1_self.T, W1_neigh.T,
                      b1.reshape(1, -1), W2_self.T, W2_neigh.T,
                      b2.reshape(1, -1))
  zeros2 = jnp.zeros((N, D_OUT), jnp.float32)
  parts2 = _AGG2(p2, src, dst, zeros2)
  return _tc2(s2, parts2, invb)


# speedup vs baseline: 11.8552x; 1.5035x over previous
import functools

import jax
import jax.numpy as jnp
from jax import lax
from jax.experimental import pallas as pl
from jax.experimental.pallas import tpu as pltpu
from jax.experimental.pallas import tpu_sc as plsc

N = 10000
E = 320000
D_IN = 128
D_HID = 150
D_OUT = 128
DF1 = 136

NC, NS = 2, 16
NW = NC * NS
EW = E // NW
K = 80
NCH = EW // K
RPS = N // NS

_MESH = plsc.VectorSubcoreMesh(
    core_axis_name="c", subcore_axis_name="s", num_cores=NC, num_subcores=NS)


def _make_agg(DF):
  @functools.partial(
      pl.kernel,
      mesh=_MESH,
      compiler_params=pltpu.CompilerParams(use_tc_tiling_on_sc=False),
      out_type=jax.ShapeDtypeStruct((NC, N, DF), jnp.float32),
      scratch_types=[
          pltpu.VMEM((NCH, K), jnp.int32),
          pltpu.VMEM((NCH, K), jnp.int32),
          pltpu.VMEM((K, DF), jnp.float32),
          pltpu.VMEM((K, DF), jnp.float32),
          pltpu.VMEM_SHARED((N, DF), jnp.float32),
          pltpu.SemaphoreType.DMA,
          pltpu.SemaphoreType.DMA,
      ],
  )
  def agg(feat_hbm, src_hbm, dst_hbm, zeros_hbm, out_hbm,
          src_v, dst_v, rows0_v, rows1_v, acc, sem0, sem1):
    c = lax.axis_index("c")
    s = lax.axis_index("s")
    wid = s * NC + c
    base = s * RPS
    pltpu.sync_copy(zeros_hbm.at[pl.ds(base, RPS)], acc.at[pl.ds(base, RPS)])
    pltpu.sync_copy(src_hbm.at[wid], src_v)
    pltpu.sync_copy(dst_hbm.at[wid], dst_v)
    plsc.subcore_barrier()

    bufs = (rows0_v, rows1_v)
    sems = (sem0, sem1)

    def gath(j, b):
      return pltpu.async_copy(feat_hbm.at[src_v.at[j]], bufs[b], sems[b])

    def scat(j, b):
      pltpu.make_async_copy(feat_hbm.at[src_v.at[j]], bufs[b], sems[b]).wait()
      pltpu.sync_copy(bufs[b], acc.at[dst_v.at[j]], add=True)

    gath(0, 0)

    def body(i, carry):
      j0 = 2 * i
      j1 = j0 + 1

      @pl.when(j1 < NCH)
      def _():
        gath(j1, 1)

      scat(j0, 0)

      @pl.when(j0 + 2 < NCH)
      def _():
        gath(j0 + 2, 0)

      @pl.when(j1 < NCH)
      def _():
        scat(j1, 1)

      return carry

    lax.fori_loop(0, (NCH + 1) // 2, body, 0)
    plsc.subcore_barrier()
    pltpu.sync_copy(acc.at[pl.ds(base, RPS)], out_hbm.at[c, pl.ds(base, RPS)])

  return agg


_AGG1 = _make_agg(DF1)
_AGG2 = _make_agg(D_OUT)

_R = 1000


def _tc1(x, parts1, w1s, w1n, b1, w2s, w2n, b2):
  def body(x_ref, p_ref, w1s_ref, w1n_ref, b1_ref, w2s_ref, w2n_ref, b2_ref,
           p2_ref, s2_ref, inv_ref):
    aggext = p_ref[0] + p_ref[1]
    inv = 1.0 / jnp.maximum(aggext[:, D_IN:D_IN + 1], 1.0)
    hn = aggext[:, :D_IN] * inv
    h1 = jnp.maximum(
        jnp.dot(x_ref[...], w1s_ref[...], preferred_element_type=jnp.float32)
        + jnp.dot(hn, w1n_ref[...], preferred_element_type=jnp.float32)
        + b1_ref[...], 0.0)
    p2_ref[...] = jnp.dot(h1, w2n_ref[...], preferred_element_type=jnp.float32)
    s2_ref[...] = (jnp.dot(h1, w2s_ref[...], preferred_element_type=jnp.float32)
                   + b2_ref[...])
    inv_ref[...] = jnp.broadcast_to(inv, (_R, D_OUT))

  return pl.pallas_call(
      body,
      grid=(N // _R,),
      in_specs=[
          pl.BlockSpec((_R, D_IN), lambda i: (i, 0)),
          pl.BlockSpec((NC, _R, DF1), lambda i: (0, i, 0)),
          pl.BlockSpec((D_IN, D_HID), lambda i: (0, 0)),
          pl.BlockSpec((D_IN, D_HID), lambda i: (0, 0)),
          pl.BlockSpec((1, D_HID), lambda i: (0, 0)),
          pl.BlockSpec((D_HID, D_OUT), lambda i: (0, 0)),
          pl.BlockSpec((D_HID, D_OUT), lambda i: (0, 0)),
          pl.BlockSpec((1, D_OUT), lambda i: (0, 0)),
      ],
      out_specs=[
          pl.BlockSpec((_R, D_OUT), lambda i: (i, 0)),
          pl.BlockSpec((_R, D_OUT), lambda i: (i, 0)),
          pl.BlockSpec((_R, D_OUT), lambda i: (i, 0)),
      ],
      out_shape=[
          jax.ShapeDtypeStruct((N, D_OUT), jnp.float32),
          jax.ShapeDtypeStruct((N, D_OUT), jnp.float32),
          jax.ShapeDtypeStruct((N, D_OUT), jnp.float32),
      ],
  )(x, parts1, w1s, w1n, b1, w2s, w2n, b2)


def _tc2(s2, parts2, invb):
  def body(s2_ref, p_ref, inv_ref, o_ref):
    o_ref[...] = s2_ref[...] + (p_ref[0] + p_ref[1]) * inv_ref[...]

  return pl.pallas_call(
      body,
      grid=(N // _R,),
      in_specs=[
          pl.BlockSpec((_R, D_OUT), lambda i: (i, 0)),
          pl.BlockSpec((NC, _R, D_OUT), lambda i: (0, i, 0)),
          pl.BlockSpec((_R, D_OUT), lambda i: (i, 0)),
      ],
      out_specs=pl.BlockSpec((_R, D_OUT), lambda i: (i, 0)),
      out_shape=jax.ShapeDtypeStruct((N, D_OUT), jnp.float32),
  )(s2, parts2, invb)


def kernel(in_feat, edge_index, W1_self, W1_neigh, b1, W2_self, W2_neigh, b2):
  src = edge_index[0].astype(jnp.int32).reshape(NW, NCH, K)
  dst = edge_index[1].astype(jnp.int32).reshape(NW, NCH, K)
  feat_ext = jnp.concatenate(
      [in_feat,
       jnp.ones((N, 1), jnp.float32),
       jnp.zeros((N, DF1 - D_IN - 1), jnp.float32)], axis=1)
  zeros1 = jnp.zeros((N, DF1), jnp.float32)
  parts1 = _AGG1(feat_ext, src, dst, zeros1)
  p2, s2, invb = _tc1(in_feat, parts1, W1_self.T, W1_neigh.T,
                      b1.reshape(1, -1), W2_self.T, W2_neigh.T,
                      b2.reshape(1, -1))
  zeros2 = jnp.zeros((N, D_OUT), jnp.float32)
  parts2 = _AGG2(p2, src, dst, zeros2)
  return _tc2(s2, parts2, invb)
